# baseline (speedup 1.0000x reference)
import functools

import jax
import jax.numpy as jnp
from jax import lax
from jax.experimental import pallas as pl
from jax.experimental.pallas import tpu as pltpu
from jax.experimental.pallas import tpu_sc as plsc

_U = jnp.uint32

_KS0 = 0
_KS1 = 42
_KS2 = 0 ^ 42 ^ 0x1BD11BDA

_ROTS = (13, 15, 26, 6, 17, 29, 16, 24, 13, 15, 26, 6, 17, 29, 16, 24, 13, 15, 26, 6)
_INJ = (
    (_KS1, (_KS2 + 1) & 0xFFFFFFFF),
    (_KS2, (_KS0 + 2) & 0xFFFFFFFF),
    (_KS0, (_KS1 + 3) & 0xFFFFFFFF),
    (_KS1, (_KS2 + 4) & 0xFFFFFFFF),
    (_KS2, None),
)


def _keep_bits(idx_u32):
    x0 = jnp.zeros_like(idx_u32)
    x1 = idx_u32 + _U(_KS1)
    for g in range(5):
        for j, r in enumerate(_ROTS[4 * g:4 * g + 4]):
            x0 = x0 + x1
            if g == 4 and j == 3:
                break
            x1 = lax.shift_left(x1, _U(r)) | lax.shift_right_logical(x1, _U(32 - r))
            x1 = x1 ^ x0
        a, b = _INJ[g]
        x0 = x0 + _U(a)
        if b is not None:
            x1 = x1 + _U(b)
    return x0


NNZ = 2684354
NW = 32
CW = 83888
MAIN31 = 83824
TAIL = NNZ - (31 * CW + MAIN31)


def _sc_body(v_hbm, o_hbm, buf, tail_buf):
    nc = 2
    wid = lax.axis_index("s") * jnp.int32(nc) + lax.axis_index("c")
    base = wid * jnp.int32(CW)

    is_last = wid == jnp.int32(NW - 1)

    @pl.when(jnp.logical_not(is_last))
    def _load_full():
        pltpu.sync_copy(v_hbm.at[pl.ds(base, CW)], buf)

    @pl.when(is_last)
    def _load_last():
        pltpu.sync_copy(v_hbm.at[pl.ds(base, MAIN31)], buf.at[pl.ds(0, MAIN31)])
        pltpu.sync_copy(v_hbm.at[pl.ds(NNZ - TAIL, TAIL)], tail_buf.at[pl.ds(0, TAIL)])

    ubase = base.astype(_U)

    def step(_, off):
        idx = lax.iota(_U, 16) + (ubase + off.astype(_U))
        keep = lax.shift_right_logical(_keep_bits(idx), _U(31)) == _U(0)
        v = buf[pl.ds(off, 16)]
        buf[pl.ds(off, 16)] = jnp.where(keep, v * 2.0, 0.0)
        return off + jnp.int32(16)

    lax.fori_loop(0, CW // 16, step, jnp.int32(0), unroll=4)

    @pl.when(is_last)
    def _tail_compute():
        idx = lax.iota(_U, 16) + _U(NNZ - TAIL)
        keep = lax.shift_right_logical(_keep_bits(idx), _U(31)) == _U(0)
        v = tail_buf[...]
        tail_buf[...] = jnp.where(keep, v * 2.0, 0.0)

    @pl.when(jnp.logical_not(is_last))
    def _store_full():
        pltpu.sync_copy(buf, o_hbm.at[pl.ds(base, CW)])

    @pl.when(is_last)
    def _store_last():
        pltpu.sync_copy(buf.at[pl.ds(0, MAIN31)], o_hbm.at[pl.ds(base, MAIN31)])
        pltpu.sync_copy(tail_buf.at[pl.ds(0, TAIL)], o_hbm.at[pl.ds(NNZ - TAIL, TAIL)])


def sc_dropout(values):
    mesh = plsc.VectorSubcoreMesh(core_axis_name="c", subcore_axis_name="s")
    return pl.kernel(
        _sc_body,
        out_type=jax.ShapeDtypeStruct((NNZ,), jnp.float32),
        mesh=mesh,
        scratch_types=[
            pltpu.VMEM((CW,), jnp.float32),
            pltpu.VMEM((16,), jnp.float32),
        ],
    )(values)


def kernel(indices, values):
    drop = sc_dropout(values)
    lo = lax.convert_element_type(indices, jnp.uint32)
    return (lo, drop)
